# Initial kernel scaffold; baseline (speedup 1.0000x reference)
import functools

import jax
import jax.numpy as jnp
from jax import lax
from jax.experimental import pallas as pl
from jax.experimental.pallas import tpu as pltpu
from jax.experimental.pallas import tpu_sc as plsc

_B = 16384
_F = 26
_D = 32
_N = _B * _F
_NC = 2
_NS = 16
_NW = _NC * _NS
_PER_W = _N // _NW
_CH = 128
_NCH = _PER_W // _CH
_LANES = 16


def _body(ids_hbm, vals_hbm, table_hbm, bias_hbm, out_hbm,
          idx_v, val_v, rows_v, bias_v, sem_t, sem_b):
    wid = lax.axis_index("s") * _NC + lax.axis_index("c")

    pltpu.sync_copy(ids_hbm.at[wid], idx_v)
    pltpu.sync_copy(vals_hbm.at[wid], val_v)

    def chunk_body(ch, carry):
        cp_t = pltpu.async_copy(table_hbm.at[idx_v.at[ch]], rows_v, sem_t)
        cp_b = pltpu.async_copy(bias_hbm.at[idx_v.at[ch]], bias_v, sem_b)
        cp_t.wait()
        cp_b.wait()

        def row_body(r, carry2):
            vexp = plsc.load_gather(
                val_v,
                [jnp.full((_LANES,), ch, jnp.int32),
                 jnp.full((_LANES,), r, jnp.int32)],
            )
            r0 = rows_v[r, pl.ds(0, _LANES)]
            r1 = rows_v[r, pl.ds(_LANES, _LANES)]
            b0 = bias_v[r, pl.ds(0, _LANES)]
            b1 = bias_v[r, pl.ds(_LANES, _LANES)]
            rows_v[r, pl.ds(0, _LANES)] = r0 * vexp + b0
            rows_v[r, pl.ds(_LANES, _LANES)] = r1 * vexp + b1
            return carry2

        lax.fori_loop(0, _CH, row_body, 0)

        base = wid * _PER_W + ch * _CH
        pltpu.sync_copy(rows_v, out_hbm.at[pl.ds(base, _CH)])
        return carry

    lax.fori_loop(0, _NCH, chunk_body, 0)


@jax.jit
def _emb(ids3, vals3, table, bias_table):
    mesh = plsc.VectorSubcoreMesh(core_axis_name="c", subcore_axis_name="s")
    f = functools.partial(
        pl.kernel,
        out_type=jax.ShapeDtypeStruct((_N, _D), jnp.float32),
        mesh=mesh,
        scratch_types=[
            pltpu.VMEM((_NCH, _CH), jnp.int32),
            pltpu.VMEM((_NCH, _CH), jnp.float32),
            pltpu.VMEM((_CH, _D), jnp.float32),
            pltpu.VMEM((_CH, _D), jnp.float32),
            pltpu.SemaphoreType.DMA,
            pltpu.SemaphoreType.DMA,
        ],
    )(_body)
    return f(ids3, vals3, table, bias_table)


def kernel(ids, values, table, bias_table):
    ids3 = ids.reshape(_NW, _NCH, _CH)
    vals3 = values.reshape(_NW, _NCH, _CH)
    out = _emb(ids3, vals3, table, bias_table)
    return out.reshape(_B, _F, _D)


# speedup vs baseline: 1.5024x; 1.5024x over previous
import functools

import jax
import jax.numpy as jnp
import numpy as np
from jax import lax
from jax.experimental import pallas as pl
from jax.experimental.pallas import tpu as pltpu
from jax.experimental.pallas import tpu_sc as plsc

_B = 16384
_F = 26
_D = 32
_N = _B * _F
_NC = 2
_NS = 16
_NW = _NC * _NS
_PER_W = _N // _NW
_CH = 128
_NCH = _PER_W // _CH
_LANES = 16


def _body(ids_hbm, vals_hbm, table_hbm, bias_hbm, out_hbm,
          idx_v, val_v, rows_v, bias_v, sem_t, sem_b):
    wid = lax.axis_index("s") * _NC + lax.axis_index("c")

    pltpu.sync_copy(ids_hbm.at[wid], idx_v)
    pltpu.sync_copy(vals_hbm.at[pl.ds(wid * _PER_W, _PER_W)], val_v)

    def chunk_body(ch, carry):
        cp_t = pltpu.async_copy(table_hbm.at[idx_v.at[ch]], rows_v, sem_t)
        cp_b = pltpu.async_copy(bias_hbm.at[idx_v.at[ch]], bias_v, sem_b)
        cp_t.wait()
        cp_b.wait()

        zero16 = lax.broadcasted_iota(jnp.int32, (_LANES,), 0) * 0

        def grp_body(g, carry2):
            val16 = val_v[pl.ds(ch * _CH + g * _LANES, _LANES)]
            for j in range(_LANES):
                row = g * _LANES + j
                vexp = val16.at[zero16 + j].get(mode="promise_in_bounds")
                r0 = rows_v[row, pl.ds(0, _LANES)]
                r1 = rows_v[row, pl.ds(_LANES, _LANES)]
                b0 = bias_v[row, pl.ds(0, _LANES)]
                b1 = bias_v[row, pl.ds(_LANES, _LANES)]
                rows_v[row, pl.ds(0, _LANES)] = r0 * vexp + b0
                rows_v[row, pl.ds(_LANES, _LANES)] = r1 * vexp + b1
            return carry2

        lax.fori_loop(0, _CH // _LANES, grp_body, 0)

        base = wid * _PER_W + ch * _CH
        pltpu.sync_copy(rows_v, out_hbm.at[pl.ds(base, _CH)])
        return carry

    lax.fori_loop(0, _NCH, chunk_body, 0)


@jax.jit
def _emb(ids3, vals3, table, bias_table):
    mesh = plsc.VectorSubcoreMesh(core_axis_name="c", subcore_axis_name="s")
    f = functools.partial(
        pl.kernel,
        out_type=jax.ShapeDtypeStruct((_N, _D), jnp.float32),
        mesh=mesh,
        compiler_params=pltpu.CompilerParams(use_tc_tiling_on_sc=False),
        scratch_types=[
            pltpu.VMEM((_NCH, _CH), jnp.int32),
            pltpu.VMEM((_PER_W,), jnp.float32),
            pltpu.VMEM((_CH, _D), jnp.float32),
            pltpu.VMEM((_CH, _D), jnp.float32),
            pltpu.SemaphoreType.DMA,
            pltpu.SemaphoreType.DMA,
        ],
    )(_body)
    return f(ids3, vals3, table, bias_table)


def kernel(ids, values, table, bias_table):
    ids3 = ids.reshape(_NW, _NCH, _CH)
    vals1 = values.reshape(_N)
    out = _emb(ids3, vals1, table, bias_table)
    return out.reshape(_B, _F, _D)


# speedup vs baseline: 1.6233x; 1.0804x over previous
import functools

import jax
import jax.numpy as jnp
from jax import lax
from jax.experimental import pallas as pl
from jax.experimental.pallas import tpu as pltpu
from jax.experimental.pallas import tpu_sc as plsc

_B = 16384
_F = 26
_D = 32
_N = _B * _F
_NC = 2
_NS = 16
_NW = _NC * _NS
_PER_W = _N // _NW
_CH = 128
_NCH = _PER_W // _CH
_LANES = 16
_NBUF = 4


def _body(ids_hbm, vals_hbm, table_hbm, bias_hbm, out_hbm,
          idx_v, val_v, rows_v, bias_v, out_v, sem_t, sem_b, sem_o):
    wid = lax.axis_index("s") * _NC + lax.axis_index("c")

    pltpu.sync_copy(ids_hbm.at[wid], idx_v)
    pltpu.sync_copy(vals_hbm.at[pl.ds(wid * _PER_W, _PER_W)], val_v)

    zero16 = lax.broadcasted_iota(jnp.int32, (_LANES,), 0) * 0

    def start_gathers(ch, b):
        pltpu.async_copy(table_hbm.at[idx_v.at[ch]], rows_v.at[b], sem_t.at[b])
        pltpu.async_copy(bias_hbm.at[idx_v.at[ch]], bias_v.at[b], sem_b.at[b])

    def wait_gathers(ch, b):
        pltpu.make_async_copy(
            table_hbm.at[idx_v.at[ch]], rows_v.at[b], sem_t.at[b]).wait()
        pltpu.make_async_copy(
            bias_hbm.at[idx_v.at[ch]], bias_v.at[b], sem_b.at[b]).wait()

    def out_slice(ch):
        return out_hbm.at[pl.ds((wid * _NCH + ch) * _CH, _CH)]

    def compute(ch, b):
        def grp_body(g, carry):
            val16 = val_v[pl.ds(ch * _CH + g * _LANES, _LANES)]
            for j in range(_LANES):
                row = g * _LANES + j
                vexp = val16.at[zero16 + j].get(mode="promise_in_bounds")
                r0 = rows_v[b, row, pl.ds(0, _LANES)]
                r1 = rows_v[b, row, pl.ds(_LANES, _LANES)]
                b0 = bias_v[b, row, pl.ds(0, _LANES)]
                b1 = bias_v[b, row, pl.ds(_LANES, _LANES)]
                out_v[b, row, pl.ds(0, _LANES)] = r0 * vexp + b0
                out_v[b, row, pl.ds(_LANES, _LANES)] = r1 * vexp + b1
            return carry

        lax.fori_loop(0, _CH // _LANES, grp_body, 0)

    for b in range(_NBUF):
        start_gathers(b, b)

    @pl.loop(0, _NCH, step=_NBUF)
    def _chunk_loop(c0):
        for b in range(_NBUF):
            ch = c0 + b

            @pl.when(ch >= _NBUF)
            def _():
                pltpu.make_async_copy(
                    out_v.at[b], out_slice(ch - _NBUF), sem_o.at[b]).wait()

            wait_gathers(ch, b)
            compute(ch, b)
            pltpu.async_copy(out_v.at[b], out_slice(ch), sem_o.at[b])

            @pl.when(ch + _NBUF < _NCH)
            def _():
                start_gathers(ch + _NBUF, b)

    for b in range(_NBUF):
        pltpu.make_async_copy(
            out_v.at[b], out_slice(_NCH - _NBUF + b), sem_o.at[b]).wait()


@jax.jit
def _emb(ids3, vals1, table, bias_table):
    mesh = plsc.VectorSubcoreMesh(core_axis_name="c", subcore_axis_name="s")
    f = functools.partial(
        pl.kernel,
        out_type=jax.ShapeDtypeStruct((_N, _D), jnp.float32),
        mesh=mesh,
        compiler_params=pltpu.CompilerParams(use_tc_tiling_on_sc=False),
        scratch_types=[
            pltpu.VMEM((_NCH, _CH), jnp.int32),
            pltpu.VMEM((_PER_W,), jnp.float32),
            pltpu.VMEM((_NBUF, _CH, _D), jnp.float32),
            pltpu.VMEM((_NBUF, _CH, _D), jnp.float32),
            pltpu.VMEM((_NBUF, _CH, _D), jnp.float32),
            pltpu.SemaphoreType.DMA((_NBUF,)),
            pltpu.SemaphoreType.DMA((_NBUF,)),
            pltpu.SemaphoreType.DMA((_NBUF,)),
        ],
    )(_body)
    return f(ids3, vals1, table, bias_table)


def kernel(ids, values, table, bias_table):
    ids3 = ids.reshape(_NW, _NCH, _CH)
    vals1 = values.reshape(_N)
    out = _emb(ids3, vals1, table, bias_table)
    return out.reshape(_B, _F, _D)
